# baseline (speedup 1.0000x reference)
import functools

import jax
import jax.numpy as jnp
from jax import lax
from jax.experimental import pallas as pl
from jax.experimental.pallas import tpu as pltpu
from jax.experimental.pallas import tpu_sc as plsc

N = 10000
E = 320000
F_IN = 128
HID = 16
HEADS = 8
NC = 64
D0 = HEADS * HID

NWORK = 32
K = 128
CHW = 80
CA128 = 102
CA64 = 84
E_PAD = NWORK * CHW * K
NA = 10112
DUMMY = 10008
RPT = 632
NEG = 0.2
EPS = 1e-16


def _attn_tail(h, a_s_ref, a_d_ref, bexp_ref, h_out, als_out, ald_out,
               m_out, wself_out, selfout_out):
    als = jnp.dot(h, a_s_ref[...], preferred_element_type=jnp.float32)
    ald = jnp.dot(h, a_d_ref[...], preferred_element_type=jnp.float32)
    m8 = jnp.maximum(
        jnp.max(als, axis=0, keepdims=True) + jnp.max(ald, axis=0, keepdims=True),
        0.0)
    m_out[...] = jnp.concatenate([m8, m8], axis=1)
    e = als + ald
    lr = jnp.where(e > 0, e, NEG * e)
    ws = jnp.exp(lr - m8)
    wself_out[...] = ws
    selfout_out[...] = h * jnp.dot(ws, bexp_ref[...],
                                   preferred_element_type=jnp.float32)
    h_out[...] = h
    als_out[...] = als
    ald_out[pl.ds(0, N), :] = ald
    ald_out[pl.ds(N, NA - N), :] = jnp.zeros((NA - N, 8), jnp.float32)


def _tc_first_body(x_ref, w_ref, a_s_ref, a_d_ref, bexp_ref,
                   h_out, als_out, ald_out, m_out, wself_out, selfout_out):
    h = jnp.dot(x_ref[...], w_ref[...], preferred_element_type=jnp.float32)
    _attn_tail(h, a_s_ref, a_d_ref, bexp_ref, h_out, als_out, ald_out,
               m_out, wself_out, selfout_out)


def _tc_mid_body(selfout_ref, wself_ref, acc_ref, den_ref, b_ref,
                 bprev_ref, w_ref, a_s_ref, a_d_ref, bexp_ref,
                 h_out, als_out, ald_out, m_out, wself_out, selfout_out):
    num = selfout_ref[...] + acc_ref[0, pl.ds(0, N), :] + acc_ref[1, pl.ds(0, N), :]
    dd = wself_ref[...] + den_ref[0, pl.ds(0, N), :] + den_ref[1, pl.ds(0, N), :]
    den_full = jnp.dot(dd, bprev_ref[...], preferred_element_type=jnp.float32)
    o = num / (den_full + EPS) + b_ref[...]
    z = jnp.where(o > 0, o, jnp.exp(o) - 1.0)
    h = jnp.dot(z, w_ref[...], preferred_element_type=jnp.float32)
    _attn_tail(h, a_s_ref, a_d_ref, bexp_ref, h_out, als_out, ald_out,
               m_out, wself_out, selfout_out)


def _tc_final_body(selfout_ref, wself_ref, acc_ref, den_ref, b_ref, bexp_ref,
                   out_ref):
    num = selfout_ref[...] + acc_ref[0, pl.ds(0, N), :] + acc_ref[1, pl.ds(0, N), :]
    dd = wself_ref[...] + den_ref[0, pl.ds(0, N), :] + den_ref[1, pl.ds(0, N), :]
    den_full = jnp.dot(dd, bexp_ref[...], preferred_element_type=jnp.float32)
    o = num / (den_full + EPS) + b_ref[...]
    mx = jnp.max(o, axis=1, keepdims=True)
    sh = o - mx
    out_ref[...] = sh - jnp.log(jnp.sum(jnp.exp(sh), axis=1, keepdims=True))


_TC_PARAMS = pltpu.CompilerParams(vmem_limit_bytes=100 * 1024 * 1024)


def _tc_first(x, w, a_s, a_d, bexp, d_out):
    return pl.pallas_call(
        _tc_first_body,
        compiler_params=_TC_PARAMS,
        out_shape=[
            jax.ShapeDtypeStruct((N, d_out), jnp.float32),
            jax.ShapeDtypeStruct((N, 8), jnp.float32),
            jax.ShapeDtypeStruct((NA, 8), jnp.float32),
            jax.ShapeDtypeStruct((1, 16), jnp.float32),
            jax.ShapeDtypeStruct((N, 8), jnp.float32),
            jax.ShapeDtypeStruct((N, d_out), jnp.float32),
        ],
    )(x, w, a_s, a_d, bexp)


def _tc_mid(selfout, wself, acc, den, b, bprev, w, a_s, a_d, bexp, d_out):
    return pl.pallas_call(
        _tc_mid_body,
        compiler_params=_TC_PARAMS,
        out_shape=[
            jax.ShapeDtypeStruct((N, d_out), jnp.float32),
            jax.ShapeDtypeStruct((N, 8), jnp.float32),
            jax.ShapeDtypeStruct((NA, 8), jnp.float32),
            jax.ShapeDtypeStruct((1, 16), jnp.float32),
            jax.ShapeDtypeStruct((N, 8), jnp.float32),
            jax.ShapeDtypeStruct((N, d_out), jnp.float32),
        ],
    )(selfout, wself, acc, den, b, bprev, w, a_s, a_d, bexp)


def _tc_final(selfout, wself, acc, den, b, bexp):
    return pl.pallas_call(
        _tc_final_body,
        compiler_params=_TC_PARAMS,
        out_shape=jax.ShapeDtypeStruct((N, NC), jnp.float32),
    )(selfout, wself, acc, den, b, bexp)


@functools.cache
def _make_sc_edge(d, ca):
    nj = d // 16
    cb = 2 * CHW - ca

    mesh = plsc.VectorSubcoreMesh(core_axis_name="c", subcore_axis_name="s")

    @functools.partial(
        pl.kernel,
        mesh=mesh,
        compiler_params=pltpu.CompilerParams(
            use_tc_tiling_on_sc=False, needs_layout_passes=False),
        out_type=(
            jax.ShapeDtypeStruct((2, NA, d), jnp.float32),
            jax.ShapeDtypeStruct((2, NA, 8), jnp.float32),
        ),
        scratch_types=[
            [pltpu.VMEM((1, K), jnp.int32)] * 2,
            [pltpu.VMEM((1, K), jnp.int32)] * 2,
            [pltpu.VMEM((K, 8), jnp.float32)] * 2,
            [pltpu.VMEM((K, 8), jnp.float32)] * 2,
            [pltpu.VMEM((K, 8), jnp.float32)] * 2,
            [pltpu.VMEM((K, d), jnp.float32)] * 2,
            pltpu.VMEM((16,), jnp.float32),
            pltpu.VMEM_SHARED((NA, d), jnp.float32),
            pltpu.VMEM_SHARED((NA, 8), jnp.float32),
            [pltpu.SemaphoreType.DMA] * 6,
        ],
    )
    def sc_edge(h_hbm, als_hbm, ald_hbm, m_hbm, src_hbm, dst_hbm,
                zacc_hbm, zden_hbm, acc_out, den_out,
                src_v, dst_v, als_v, ald_v, w_v, h_v, m_v,
                acc_s, den_s, sems):
        c = lax.axis_index("c")
        s = lax.axis_index("s")
        chw = jnp.where(c == 0, ca, cb)
        wbase = c * (16 * ca) + s * chw

        pltpu.sync_copy(zacc_hbm, acc_s.at[pl.ds(s * RPT, RPT)])
        pltpu.sync_copy(zden_hbm, den_s.at[pl.ds(s * RPT, RPT)])
        pltpu.sync_copy(m_hbm.at[0], m_v)
        plsc.subcore_barrier()

        iota = lax.iota(jnp.int32, 16)
        row_pat = iota // 8
        col_pat = iota - row_pat * 8
        mvec = m_v[...]
        splat_vecs = [jnp.full((16,), j, jnp.int32) for j in range(16)]

        def fire(b, cc):
            row = wbase + cc
            pltpu.sync_copy(src_hbm.at[pl.ds(row, 1)], src_v[b])
            pltpu.sync_copy(dst_hbm.at[pl.ds(row, 1)], dst_v[b])
            pltpu.async_copy(als_hbm.at[src_v[b].at[0]], als_v[b], sems[3 * b])
            pltpu.async_copy(ald_hbm.at[dst_v[b].at[0]], ald_v[b], sems[3 * b + 1])
            pltpu.async_copy(h_hbm.at[src_v[b].at[0]], h_v[b], sems[3 * b + 2])

        def drain(b):
            pltpu.make_async_copy(als_hbm.at[src_v[b].at[0]], als_v[b],
                                  sems[3 * b]).wait()
            pltpu.make_async_copy(ald_hbm.at[dst_v[b].at[0]], ald_v[b],
                                  sems[3 * b + 1]).wait()
            pltpu.make_async_copy(h_hbm.at[src_v[b].at[0]], h_v[b],
                                  sems[3 * b + 2]).wait()

        def process(b):
            alsb, aldb, wb, hb = als_v[b], ald_v[b], w_v[b], h_v[b]

            @pl.loop(0, K // 2, unroll=4)
            def _floop(i):
                r = 2 * i + row_pat
                av = plsc.load_gather(alsb, [r, col_pat])
                dv = plsc.load_gather(aldb, [r, col_pat])
                e = av + dv
                lr = jnp.where(e > 0, e, NEG * e)
                w = jnp.exp(lr - mvec)
                plsc.store_scatter(wb, [r, col_pat], w)
                for half in range(2):
                    ei = 2 * i + half
                    for j in range(nj):
                        wsp = w.at[splat_vecs[half * 8 + j]].get(
                            mode="promise_in_bounds")
                        sl = pl.ds(j * 16, 16)
                        hb[ei, sl] = hb[ei, sl] * wsp

            pltpu.sync_copy(wb, den_s.at[dst_v[b].at[0]], add=True)
            pltpu.sync_copy(hb, acc_s.at[dst_v[b].at[0]], add=True)

        fire(0, 0)

        @pl.loop(0, chw // 2)
        def _pair(p):
            fire(1, 2 * p + 1)
            drain(0)
            process(0)
            fire(0, jnp.minimum(2 * p + 2, chw - 1))
            drain(1)
            process(1)

        drain(0)

        plsc.subcore_barrier()
        sl = pl.ds(s * RPT, RPT)
        pltpu.sync_copy(acc_s.at[sl], acc_out.at[c].at[sl])
        pltpu.sync_copy(den_s.at[sl], den_out.at[c].at[sl])

    return sc_edge


def _expand_attn(a):
    h, c = a.shape
    eye = jnp.eye(8, dtype=jnp.float32)[:h]
    return (a[:, :, None] * eye[:, None, :]).reshape(h * c, 8)


def kernel(x, edge_index, W0, a_src0, a_dst0, b0, W1, a_src1, a_dst1, b1,
           W2, a_src2, a_dst2, b2):
    ei = edge_index.astype(jnp.int32)
    pad = E_PAD - E
    src = jnp.concatenate([ei[0], jnp.zeros((pad,), jnp.int32)])
    dst = jnp.concatenate([ei[1], jnp.full((pad,), DUMMY, jnp.int32)])
    src2d = src.reshape(E_PAD // 128, 128)
    dst2d = dst.reshape(E_PAD // 128, 128)

    a_s0 = _expand_attn(a_src0)
    a_d0 = _expand_attn(a_dst0)
    a_s1 = _expand_attn(a_src1)
    a_d1 = _expand_attn(a_dst1)
    a_s2 = a_src2.reshape(NC, 1) * jnp.ones((1, 8), jnp.float32)
    a_d2 = a_dst2.reshape(NC, 1) * jnp.ones((1, 8), jnp.float32)
    bexp0 = jnp.kron(jnp.eye(8, dtype=jnp.float32), jnp.ones((1, 16), jnp.float32))
    bexp2 = jnp.zeros((8, NC), jnp.float32).at[0].set(1.0)
    z128 = jnp.zeros((RPT, D0), jnp.float32)
    z64 = jnp.zeros((RPT, NC), jnp.float32)
    z8 = jnp.zeros((RPT, 8), jnp.float32)
    b0r = b0.reshape(1, D0)
    b1r = b1.reshape(1, D0)
    b2r = b2.reshape(1, NC)

    sc128 = _make_sc_edge(D0, CA128)
    sc64 = _make_sc_edge(NC, CA64)

    h0, als0, ald0, m0, ws0, so0 = _tc_first(x, W0, a_s0, a_d0, bexp0, D0)
    acc0, den0 = sc128(h0, als0, ald0, m0, src2d, dst2d, z128, z8)

    h1, als1, ald1, m1, ws1, so1 = _tc_mid(
        so0, ws0, acc0, den0, b0r, bexp0, W1, a_s1, a_d1, bexp0, D0)
    acc1, den1 = sc128(h1, als1, ald1, m1, src2d, dst2d, z128, z8)

    h2, als2, ald2, m2, ws2, so2 = _tc_mid(
        so1, ws1, acc1, den1, b1r, bexp0, W2, a_s2, a_d2, bexp2, NC)
    acc2, den2 = sc64(h2, als2, ald2, m2, src2d, dst2d, z64, z8)

    return _tc_final(so2, ws2, acc2, den2, b2r, bexp2)


# speedup vs baseline: 1.3928x; 1.3928x over previous
import functools

import jax
import jax.numpy as jnp
from jax import lax
from jax.experimental import pallas as pl
from jax.experimental.pallas import tpu as pltpu
from jax.experimental.pallas import tpu_sc as plsc

N = 10000
E = 320000
F_IN = 128
HID = 16
HEADS = 8
NC = 64
D0 = HEADS * HID

NWORK = 32
K = 128
CHW = 80
CA128 = 102
CA64 = 84
E_PAD = NWORK * CHW * K
NA = 10112
DUMMY = 10008
RPT = 632
NEG = 0.2
EPS = 1e-16


def _attn_tail(h, a_s_ref, a_d_ref, bexp_ref, h_out, als_out, ald_out,
               m_out, wself_out, selfout_out):
    als = jnp.dot(h, a_s_ref[...], preferred_element_type=jnp.float32)
    ald = jnp.dot(h, a_d_ref[...], preferred_element_type=jnp.float32)
    m8 = jnp.maximum(
        jnp.max(als, axis=0, keepdims=True) + jnp.max(ald, axis=0, keepdims=True),
        0.0)
    m_out[...] = jnp.concatenate([m8, m8], axis=1)
    e = als + ald
    lr = jnp.where(e > 0, e, NEG * e)
    ws = jnp.exp(lr - m8)
    wself_out[...] = ws
    selfout_out[...] = h * jnp.dot(ws, bexp_ref[...],
                                   preferred_element_type=jnp.float32)
    h_out[...] = h
    als_out[...] = als
    ald_out[pl.ds(0, N), :] = ald
    ald_out[pl.ds(N, NA - N), :] = jnp.zeros((NA - N, 8), jnp.float32)


def _tc_first_body(x_ref, w_ref, a_s_ref, a_d_ref, bexp_ref,
                   h_out, als_out, ald_out, m_out, wself_out, selfout_out):
    h = jnp.dot(x_ref[...], w_ref[...], preferred_element_type=jnp.float32)
    _attn_tail(h, a_s_ref, a_d_ref, bexp_ref, h_out, als_out, ald_out,
               m_out, wself_out, selfout_out)


def _tc_mid_body(selfout_ref, wself_ref, acc_ref, den_ref, b_ref,
                 bprev_ref, w_ref, a_s_ref, a_d_ref, bexp_ref,
                 h_out, als_out, ald_out, m_out, wself_out, selfout_out):
    num = selfout_ref[...] + acc_ref[0, pl.ds(0, N), :] + acc_ref[1, pl.ds(0, N), :]
    dd = wself_ref[...] + den_ref[0, pl.ds(0, N), :] + den_ref[1, pl.ds(0, N), :]
    den_full = jnp.dot(dd, bprev_ref[...], preferred_element_type=jnp.float32)
    o = num / (den_full + EPS) + b_ref[...]
    z = jnp.where(o > 0, o, jnp.exp(o) - 1.0)
    h = jnp.dot(z, w_ref[...], preferred_element_type=jnp.float32)
    _attn_tail(h, a_s_ref, a_d_ref, bexp_ref, h_out, als_out, ald_out,
               m_out, wself_out, selfout_out)


def _tc_final_body(selfout_ref, wself_ref, acc_ref, den_ref, b_ref, bexp_ref,
                   out_ref):
    num = selfout_ref[...] + acc_ref[0, pl.ds(0, N), :] + acc_ref[1, pl.ds(0, N), :]
    dd = wself_ref[...] + den_ref[0, pl.ds(0, N), :] + den_ref[1, pl.ds(0, N), :]
    den_full = jnp.dot(dd, bexp_ref[...], preferred_element_type=jnp.float32)
    o = num / (den_full + EPS) + b_ref[...]
    mx = jnp.max(o, axis=1, keepdims=True)
    sh = o - mx
    out_ref[...] = sh - jnp.log(jnp.sum(jnp.exp(sh), axis=1, keepdims=True))


_TC_PARAMS = pltpu.CompilerParams(vmem_limit_bytes=100 * 1024 * 1024)


def _tc_first(x, w, a_s, a_d, bexp, d_out):
    return pl.pallas_call(
        _tc_first_body,
        compiler_params=_TC_PARAMS,
        out_shape=[
            jax.ShapeDtypeStruct((N, d_out), jnp.float32),
            jax.ShapeDtypeStruct((N, 8), jnp.float32),
            jax.ShapeDtypeStruct((NA, 8), jnp.float32),
            jax.ShapeDtypeStruct((1, 16), jnp.float32),
            jax.ShapeDtypeStruct((N, 8), jnp.float32),
            jax.ShapeDtypeStruct((N, d_out), jnp.float32),
        ],
    )(x, w, a_s, a_d, bexp)


def _tc_mid(selfout, wself, acc, den, b, bprev, w, a_s, a_d, bexp, d_out):
    return pl.pallas_call(
        _tc_mid_body,
        compiler_params=_TC_PARAMS,
        out_shape=[
            jax.ShapeDtypeStruct((N, d_out), jnp.float32),
            jax.ShapeDtypeStruct((N, 8), jnp.float32),
            jax.ShapeDtypeStruct((NA, 8), jnp.float32),
            jax.ShapeDtypeStruct((1, 16), jnp.float32),
            jax.ShapeDtypeStruct((N, 8), jnp.float32),
            jax.ShapeDtypeStruct((N, d_out), jnp.float32),
        ],
    )(selfout, wself, acc, den, b, bprev, w, a_s, a_d, bexp)


def _tc_final(selfout, wself, acc, den, b, bexp):
    return pl.pallas_call(
        _tc_final_body,
        compiler_params=_TC_PARAMS,
        out_shape=jax.ShapeDtypeStruct((N, NC), jnp.float32),
    )(selfout, wself, acc, den, b, bexp)


@functools.cache
def _make_sc_edge(d, ca):
    nj = d // 16
    cb = 2 * CHW - ca

    mesh = plsc.VectorSubcoreMesh(core_axis_name="c", subcore_axis_name="s")

    @functools.partial(
        pl.kernel,
        mesh=mesh,
        compiler_params=pltpu.CompilerParams(
            use_tc_tiling_on_sc=False, needs_layout_passes=False),
        out_type=(
            jax.ShapeDtypeStruct((2, NA, d), jnp.float32),
            jax.ShapeDtypeStruct((2, NA, 8), jnp.float32),
        ),
        scratch_types=[
            [pltpu.VMEM((1, K), jnp.int32)] * 2,
            [pltpu.VMEM((1, K), jnp.int32)] * 2,
            [pltpu.VMEM((K, 8), jnp.float32)] * 2,
            [pltpu.VMEM((K, 8), jnp.float32)] * 2,
            [pltpu.VMEM((K, 8), jnp.float32)] * 2,
            [pltpu.VMEM((K, d), jnp.float32)] * 2,
            pltpu.VMEM((16,), jnp.float32),
            pltpu.VMEM_SHARED((NA, d), jnp.float32),
            pltpu.VMEM_SHARED((NA, 8), jnp.float32),
            [pltpu.SemaphoreType.DMA] * 6,
        ],
    )
    def sc_edge(h_hbm, als_hbm, ald_hbm, m_hbm, src_hbm, dst_hbm,
                zacc_hbm, zden_hbm, acc_out, den_out,
                src_v, dst_v, als_v, ald_v, w_v, h_v, m_v,
                acc_s, den_s, sems):
        c = lax.axis_index("c")
        s = lax.axis_index("s")
        chw = jnp.where(c == 0, ca, cb)
        wbase = c * (16 * ca) + s * chw

        pltpu.sync_copy(zacc_hbm, acc_s.at[pl.ds(s * RPT, RPT)])
        pltpu.sync_copy(zden_hbm, den_s.at[pl.ds(s * RPT, RPT)])
        pltpu.sync_copy(m_hbm.at[0], m_v)
        plsc.subcore_barrier()

        iota = lax.iota(jnp.int32, 16)
        row_pat = iota // 8
        col_pat = iota - row_pat * 8
        mvec = m_v[...]
        splat_vecs = [jnp.full((16,), j, jnp.int32) for j in range(16)]

        def fire(b, cc):
            row = wbase + cc
            pltpu.sync_copy(src_hbm.at[pl.ds(row, 1)], src_v[b])
            pltpu.sync_copy(dst_hbm.at[pl.ds(row, 1)], dst_v[b])
            pltpu.async_copy(als_hbm.at[src_v[b].at[0]], als_v[b], sems[3 * b])
            pltpu.async_copy(ald_hbm.at[dst_v[b].at[0]], ald_v[b], sems[3 * b + 1])
            pltpu.async_copy(h_hbm.at[src_v[b].at[0]], h_v[b], sems[3 * b + 2])

        def drain(b):
            pltpu.make_async_copy(als_hbm.at[src_v[b].at[0]], als_v[b],
                                  sems[3 * b]).wait()
            pltpu.make_async_copy(ald_hbm.at[dst_v[b].at[0]], ald_v[b],
                                  sems[3 * b + 1]).wait()
            pltpu.make_async_copy(h_hbm.at[src_v[b].at[0]], h_v[b],
                                  sems[3 * b + 2]).wait()

        def process(b):
            alsb, aldb, wb, hb = als_v[b], ald_v[b], w_v[b], h_v[b]

            @pl.loop(0, K // 2, unroll=2)
            def _floop(i):
                r = 2 * i + row_pat
                av = plsc.load_gather(alsb, [r, col_pat])
                dv = plsc.load_gather(aldb, [r, col_pat])
                e = av + dv
                lr = jnp.where(e > 0, e, NEG * e)
                w = jnp.exp(lr - mvec)
                plsc.store_scatter(wb, [r, col_pat], w)
                for half in range(2):
                    ei = 2 * i + half
                    for j in range(nj):
                        wsp = w.at[splat_vecs[half * 8 + j]].get(
                            mode="promise_in_bounds")
                        sl = pl.ds(j * 16, 16)
                        hb[ei, sl] = hb[ei, sl] * wsp

            pltpu.sync_copy(wb, den_s.at[dst_v[b].at[0]], add=True)
            pltpu.sync_copy(hb, acc_s.at[dst_v[b].at[0]], add=True)

        fire(0, 0)

        @pl.loop(0, chw // 2)
        def _pair(p):
            fire(1, 2 * p + 1)
            drain(0)
            process(0)
            fire(0, jnp.minimum(2 * p + 2, chw - 1))
            drain(1)
            process(1)

        drain(0)

        plsc.subcore_barrier()
        sl = pl.ds(s * RPT, RPT)
        pltpu.sync_copy(acc_s.at[sl], acc_out.at[c].at[sl])
        pltpu.sync_copy(den_s.at[sl], den_out.at[c].at[sl])

    return sc_edge


def _expand_attn(a):
    h, c = a.shape
    eye = jnp.eye(8, dtype=jnp.float32)[:h]
    return (a[:, :, None] * eye[:, None, :]).reshape(h * c, 8)


def kernel(x, edge_index, W0, a_src0, a_dst0, b0, W1, a_src1, a_dst1, b1,
           W2, a_src2, a_dst2, b2):
    ei = edge_index.astype(jnp.int32)
    pad = E_PAD - E
    src = jnp.concatenate([ei[0], jnp.zeros((pad,), jnp.int32)])
    dst = jnp.concatenate([ei[1], jnp.full((pad,), DUMMY, jnp.int32)])
    src2d = src.reshape(E_PAD // 128, 128)
    dst2d = dst.reshape(E_PAD // 128, 128)

    a_s0 = _expand_attn(a_src0)
    a_d0 = _expand_attn(a_dst0)
    a_s1 = _expand_attn(a_src1)
    a_d1 = _expand_attn(a_dst1)
    a_s2 = a_src2.reshape(NC, 1) * jnp.ones((1, 8), jnp.float32)
    a_d2 = a_dst2.reshape(NC, 1) * jnp.ones((1, 8), jnp.float32)
    bexp0 = jnp.kron(jnp.eye(8, dtype=jnp.float32), jnp.ones((1, 16), jnp.float32))
    bexp2 = jnp.zeros((8, NC), jnp.float32).at[0].set(1.0)
    z128 = jnp.zeros((RPT, D0), jnp.float32)
    z64 = jnp.zeros((RPT, NC), jnp.float32)
    z8 = jnp.zeros((RPT, 8), jnp.float32)
    b0r = b0.reshape(1, D0)
    b1r = b1.reshape(1, D0)
    b2r = b2.reshape(1, NC)

    sc128 = _make_sc_edge(D0, CA128)
    sc64 = _make_sc_edge(NC, CA64)

    h0, als0, ald0, m0, ws0, so0 = _tc_first(x, W0, a_s0, a_d0, bexp0, D0)
    acc0, den0 = sc128(h0, als0, ald0, m0, src2d, dst2d, z128, z8)

    h1, als1, ald1, m1, ws1, so1 = _tc_mid(
        so0, ws0, acc0, den0, b0r, bexp0, W1, a_s1, a_d1, bexp0, D0)
    acc1, den1 = sc128(h1, als1, ald1, m1, src2d, dst2d, z128, z8)

    h2, als2, ald2, m2, ws2, so2 = _tc_mid(
        so1, ws1, acc1, den1, b1r, bexp0, W2, a_s2, a_d2, bexp2, NC)
    acc2, den2 = sc64(h2, als2, ald2, m2, src2d, dst2d, z64, z8)

    return _tc_final(so2, ws2, acc2, den2, b2r, bexp2)
